# baseline (speedup 1.0000x reference)
import functools

import jax
import jax.numpy as jnp
from jax import lax
from jax.experimental import pallas as pl
from jax.experimental.pallas import tpu as pltpu
from jax.experimental.pallas import tpu_sc as plsc

N = 10000
E = 320000
D_IN = 128
H1 = 64

NC = 2
NS = 16
NW = NC * NS
NPAD = 10240
ROWS_PER_TILE = NPAD // NS
EPT = E // NW
CHUNK = 128
NBUF = 2
NCHUNK = 80
EPT_PAD = NCHUNK * CHUNK
PADROW = NPAD - 1

_mesh = plsc.VectorSubcoreMesh(core_axis_name="c", subcore_axis_name="s")
_sc_params = pltpu.CompilerParams(
    needs_layout_passes=False, use_tc_tiling_on_sc=False)


@functools.partial(
    pl.kernel,
    out_type=jax.ShapeDtypeStruct((NW, NPAD), jnp.float32),
    mesh=_mesh,
    compiler_params=_sc_params,
    scratch_types=[
        pltpu.VMEM((EPT_PAD,), jnp.int32),
        pltpu.VMEM((NPAD,), jnp.float32),
    ],
)
def _deg_kernel(dst_hbm, degp_hbm, idx_v, hist_v):
    c = lax.axis_index("c")
    s = lax.axis_index("s")
    wid = s * NC + c
    pltpu.sync_copy(dst_hbm.at[wid], idx_v)

    def zero_body(i, _):
        hist_v[pl.ds(i * 16, 16)] = jnp.zeros((16,), jnp.float32)
        return 0

    lax.fori_loop(0, NPAD // 16, zero_body, 0)
    ones = jnp.ones((16,), jnp.float32)

    def body(g, _):
        idx = idx_v[pl.ds(g * 16, 16)]
        plsc.addupdate_scatter(hist_v, [idx], ones)
        return 0

    lax.fori_loop(0, EPT_PAD // 16, body, 0)
    pltpu.sync_copy(hist_v, degp_hbm.at[wid])


@functools.partial(
    pl.kernel,
    out_type=jax.ShapeDtypeStruct((NC, NPAD, H1), jnp.float32),
    mesh=_mesh,
    compiler_params=_sc_params,
    scratch_types=[
        pltpu.VMEM((NCHUNK, CHUNK), jnp.int32),
        pltpu.VMEM((NCHUNK, CHUNK), jnp.int32),
    ] + [pltpu.VMEM((CHUNK, H1), jnp.float32) for _ in range(NBUF)]
      + [pltpu.SemaphoreType.DMA for _ in range(2 * NBUF)]
      + [pltpu.VMEM_SHARED((NPAD, H1), jnp.float32),
         pltpu.VMEM_SHARED((NPAD, H1), jnp.float32)],
)
def _agg_kernel(u_hbm, uh_hbm, src_hbm, dst_hbm, out_hbm, src_v, dst_v, *scr):
    rows = scr[0:NBUF]
    gsem = scr[NBUF:2 * NBUF]
    ssem = scr[2 * NBUF:3 * NBUF]
    acc = scr[3 * NBUF]
    u_s = scr[3 * NBUF + 1]
    c = lax.axis_index("c")
    s = lax.axis_index("s")
    wid = s * NC + c
    pltpu.sync_copy(src_hbm.at[wid], src_v)
    pltpu.sync_copy(dst_hbm.at[wid], dst_v)
    base = s * ROWS_PER_TILE
    pltpu.sync_copy(u_hbm.at[pl.ds(base, ROWS_PER_TILE)],
                    u_s.at[pl.ds(base, ROWS_PER_TILE)])
    pltpu.sync_copy(uh_hbm.at[pl.ds(base, ROWS_PER_TILE)],
                    acc.at[pl.ds(base, ROWS_PER_TILE)])
    plsc.subcore_barrier()

    for b in range(NBUF):
        pltpu.async_copy(u_s.at[src_v.at[b]], rows[b], gsem[b])

    def outer(t, _):
        cbase = t * NBUF
        for b in range(NBUF):
            j = cbase + b
            pltpu.make_async_copy(u_s.at[src_v.at[j]], rows[b], gsem[b]).wait()
            pltpu.async_copy(rows[b], acc.at[dst_v.at[j]], ssem[b], add=True)
        for b in range(NBUF):
            j = cbase + b

            @pl.when(j + NBUF < NCHUNK)
            def _():
                pltpu.make_async_copy(
                    rows[b], acc.at[dst_v.at[j]], ssem[b]).wait()
                pltpu.async_copy(u_s.at[src_v.at[j + NBUF]], rows[b], gsem[b])
        return 0

    lax.fori_loop(0, NCHUNK // NBUF, outer, 0)
    for b in range(NBUF):
        pltpu.make_async_copy(
            rows[b], acc.at[dst_v.at[NCHUNK - NBUF + b]], ssem[b]).wait()
    plsc.subcore_barrier()
    pltpu.sync_copy(acc.at[pl.ds(base, ROWS_PER_TILE)],
                    out_hbm.at[c, pl.ds(base, ROWS_PER_TILE)])


BLK = 512
GRID = NPAD // BLK


def _k2_body(x_ref, w0_ref, degp_ref, u0_ref, u0h_ref, dinv_ref):
    deg = jnp.sum(degp_ref[...], axis=0) + 1.0
    dinv = lax.rsqrt(deg)
    y0 = jnp.dot(x_ref[...], w0_ref[...], preferred_element_type=jnp.float32)
    u0 = y0 * dinv[:, None]
    u0_ref[...] = u0
    u0h_ref[...] = 0.5 * u0
    dinv_ref[...] = dinv


def _k4_body(p_ref, dinv_ref, u1_ref, u1h_ref):
    p = p_ref[...]
    d = dinv_ref[...]
    u1 = (d * d)[:, None] * (p[0] + p[1])
    u1_ref[...] = u1
    u1h_ref[...] = 0.5 * u1


def _sigmoid(v):
    return 1.0 / (1.0 + jnp.exp(-v))


def _softplus(v):
    return jnp.maximum(v, 0.0) + jnp.log(1.0 + jnp.exp(-jnp.abs(v)))


def _k5_body(q_ref, dinv_ref, wcat_ref, fblk_ref, zm_ref, zs_ref, zp_ref, za_ref):
    q = q_ref[...]
    g = dinv_ref[...][:, None] * (q[0] + q[1])
    h = _sigmoid(jnp.dot(g, wcat_ref[...], preferred_element_type=jnp.float32))
    z = jnp.dot(h, fblk_ref[...], preferred_element_type=jnp.float32)
    zm_ref[...] = z[:, 0:32]
    zs_ref[...] = z[:, 32:64]
    zp_ref[...] = z[:, 64:96]
    za_ref[...] = _softplus(z[:, 96:128])


def kernel(x, edge_index, W0, Wm, Ws, Wp, Wa, Fm, Fs, Fp, Fa):
    src = edge_index[0]
    dst = edge_index[1]
    src_t = jnp.pad(src.reshape(NW, EPT), ((0, 0), (0, EPT_PAD - EPT)),
                    constant_values=PADROW).reshape(NW, NCHUNK, CHUNK)
    dst_t = jnp.pad(dst.reshape(NW, EPT), ((0, 0), (0, EPT_PAD - EPT)),
                    constant_values=PADROW).reshape(NW, NCHUNK, CHUNK)
    dst_f = dst_t.reshape(NW, EPT_PAD)
    x_pad = jnp.pad(x, ((0, NPAD - N), (0, 0)))
    wcat = jnp.concatenate([Wm, Ws, Wp, Wa], axis=1)
    fblk = jax.scipy.linalg.block_diag(Fm, Fs, Fp, Fa)

    degp = _deg_kernel(dst_f)

    u0, u0h, dinv = pl.pallas_call(
        _k2_body,
        grid=(GRID,),
        in_specs=[
            pl.BlockSpec((BLK, D_IN), lambda i: (i, 0)),
            pl.BlockSpec((D_IN, H1), lambda i: (0, 0)),
            pl.BlockSpec((NW, BLK), lambda i: (0, i)),
        ],
        out_specs=[
            pl.BlockSpec((BLK, H1), lambda i: (i, 0)),
            pl.BlockSpec((BLK, H1), lambda i: (i, 0)),
            pl.BlockSpec((BLK,), lambda i: (i,)),
        ],
        out_shape=[
            jax.ShapeDtypeStruct((NPAD, H1), jnp.float32),
            jax.ShapeDtypeStruct((NPAD, H1), jnp.float32),
            jax.ShapeDtypeStruct((NPAD,), jnp.float32),
        ],
    )(x_pad, W0, degp)

    p = _agg_kernel(u0, u0h, src_t, dst_t)

    u1, u1h = pl.pallas_call(
        _k4_body,
        grid=(GRID,),
        in_specs=[
            pl.BlockSpec((NC, BLK, H1), lambda i: (0, i, 0)),
            pl.BlockSpec((BLK,), lambda i: (i,)),
        ],
        out_specs=[
            pl.BlockSpec((BLK, H1), lambda i: (i, 0)),
            pl.BlockSpec((BLK, H1), lambda i: (i, 0)),
        ],
        out_shape=[
            jax.ShapeDtypeStruct((NPAD, H1), jnp.float32),
            jax.ShapeDtypeStruct((NPAD, H1), jnp.float32),
        ],
    )(p, dinv)

    q = _agg_kernel(u1, u1h, src_t, dst_t)

    zm, zs, zp, za = pl.pallas_call(
        _k5_body,
        grid=(GRID,),
        in_specs=[
            pl.BlockSpec((NC, BLK, H1), lambda i: (0, i, 0)),
            pl.BlockSpec((BLK,), lambda i: (i,)),
            pl.BlockSpec((H1, 128), lambda i: (0, 0)),
            pl.BlockSpec((128, 128), lambda i: (0, 0)),
        ],
        out_specs=[pl.BlockSpec((BLK, 32), lambda i: (i, 0))] * 4,
        out_shape=[jax.ShapeDtypeStruct((NPAD, 32), jnp.float32)] * 4,
    )(q, dinv, wcat, fblk)

    return jnp.stack([zm, zs, zp, za])[:, :N]


# speedup vs baseline: 1.0714x; 1.0714x over previous
import functools

import jax
import jax.numpy as jnp
from jax import lax
from jax.experimental import pallas as pl
from jax.experimental.pallas import tpu as pltpu
from jax.experimental.pallas import tpu_sc as plsc

N = 10000
E = 320000
D_IN = 128
H1 = 64
HC = 32

NC = 2
NS = 16
NW = NC * NS
NPAD = 10240
ROWS_PER_TILE = NPAD // NS
RBLK = 128
NRBLK = ROWS_PER_TILE // RBLK

EPT = E // NS
CHUNK = 128
NBUF = 2
NCHUNK = 160
EPT_PAD = NCHUNK * CHUNK
PADROW = NPAD - 1

_mesh = plsc.VectorSubcoreMesh(core_axis_name="c", subcore_axis_name="s")
_sc_params = pltpu.CompilerParams(
    needs_layout_passes=False, use_tc_tiling_on_sc=False)


@functools.partial(
    pl.kernel,
    out_type=jax.ShapeDtypeStruct((NW, NPAD), jnp.float32),
    mesh=_mesh,
    compiler_params=_sc_params,
    scratch_types=[
        pltpu.VMEM((EPT_PAD // 2,), jnp.int32),
        pltpu.VMEM((NPAD,), jnp.float32),
    ],
)
def _deg_kernel(dst_hbm, degp_hbm, idx_v, hist_v):
    c = lax.axis_index("c")
    s = lax.axis_index("s")
    wid = s * NC + c
    pltpu.sync_copy(dst_hbm.at[wid], idx_v)

    def zero_body(i, _):
        hist_v[pl.ds(i * 16, 16)] = jnp.zeros((16,), jnp.float32)
        return 0

    lax.fori_loop(0, NPAD // 16, zero_body, 0)
    ones = jnp.ones((16,), jnp.float32)

    def body(g, _):
        idx = idx_v[pl.ds(g * 16, 16)]
        plsc.addupdate_scatter(hist_v, [idx], ones)
        return 0

    lax.fori_loop(0, EPT_PAD // 2 // 16, body, 0)
    pltpu.sync_copy(hist_v, degp_hbm.at[wid])


def _ewise_mul(dst_ref, a_ref, b_ref):
    def body(r, _):
        for o in range(HC // 16):
            sl = pl.ds(o * 16, 16)
            dst_ref[r, sl] = a_ref[r, sl] * b_ref[r, sl]
        return 0

    lax.fori_loop(0, RBLK, body, 0)


@functools.partial(
    pl.kernel,
    out_type=jax.ShapeDtypeStruct((NPAD, H1), jnp.float32),
    mesh=_mesh,
    compiler_params=_sc_params,
    scratch_types=[
        pltpu.VMEM((NCHUNK, CHUNK), jnp.int32),
        pltpu.VMEM((NCHUNK, CHUNK), jnp.int32),
    ] + [pltpu.VMEM((CHUNK, HC), jnp.float32) for _ in range(NBUF)]
      + [pltpu.VMEM((RBLK, HC), jnp.float32)]
      + [pltpu.SemaphoreType.DMA for _ in range(2 * NBUF)]
      + [pltpu.VMEM_SHARED((NPAD, HC), jnp.float32),
         pltpu.VMEM_SHARED((NPAD, HC), jnp.float32)],
)
def _gcn_kernel(u0_hbm, d2b_hbm, db_hbm, src_hbm, dst_hbm, g1_hbm, src_v,
                dst_v, *scr):
    rows = scr[0:NBUF]
    aux = scr[NBUF]
    gsem = scr[NBUF + 1:2 * NBUF + 1]
    ssem = scr[2 * NBUF + 1:3 * NBUF + 1]
    acc = scr[3 * NBUF + 1]
    u_s = scr[3 * NBUF + 2]
    c = lax.axis_index("c")
    s = lax.axis_index("s")
    base = s * ROWS_PER_TILE
    cols = pl.ds(c * HC, HC)

    pltpu.sync_copy(src_hbm.at[s], src_v)
    pltpu.sync_copy(dst_hbm.at[s], dst_v)
    rsl = pl.ds(base, ROWS_PER_TILE)
    pltpu.sync_copy(u0_hbm.at[rsl, cols], u_s.at[rsl])
    pltpu.sync_copy(u0_hbm.at[rsl, cols], acc.at[rsl])
    plsc.subcore_barrier()

    def edge_pass():
        for b in range(NBUF):
            pltpu.async_copy(u_s.at[src_v.at[b]], rows[b], gsem[b])

        def outer(t, _):
            cbase = t * NBUF
            for b in range(NBUF):
                j = cbase + b
                pltpu.make_async_copy(
                    u_s.at[src_v.at[j]], rows[b], gsem[b]).wait()
                pltpu.async_copy(rows[b], acc.at[dst_v.at[j]], ssem[b],
                                 add=True)
            for b in range(NBUF):
                j = cbase + b

                @pl.when(j + NBUF < NCHUNK)
                def _():
                    pltpu.make_async_copy(
                        rows[b], acc.at[dst_v.at[j]], ssem[b]).wait()
                    pltpu.async_copy(
                        u_s.at[src_v.at[j + NBUF]], rows[b], gsem[b])
            return 0

        lax.fori_loop(0, NCHUNK // NBUF, outer, 0)
        for b in range(NBUF):
            pltpu.make_async_copy(
                rows[b], acc.at[dst_v.at[NCHUNK - NBUF + b]], ssem[b]).wait()

    edge_pass()
    plsc.subcore_barrier()

    for k in range(NRBLK):
        rk = pl.ds(base + k * RBLK, RBLK)
        pltpu.sync_copy(acc.at[rk], rows[0])
        pltpu.sync_copy(d2b_hbm.at[rk], aux)
        _ewise_mul(rows[1], rows[0], aux)
        pltpu.sync_copy(rows[1], u_s.at[rk])
        pltpu.sync_copy(rows[1], acc.at[rk])
    plsc.subcore_barrier()

    edge_pass()
    plsc.subcore_barrier()

    for k in range(NRBLK):
        rk = pl.ds(base + k * RBLK, RBLK)
        pltpu.sync_copy(acc.at[rk], rows[0])
        pltpu.sync_copy(db_hbm.at[rk], aux)
        _ewise_mul(rows[1], rows[0], aux)
        pltpu.sync_copy(rows[1], g1_hbm.at[rk, cols])


BLK = 512
GRID = NPAD // BLK


def _k2_body(x_ref, w0_ref, degp_ref, u0_ref, d2b_ref, db_ref):
    deg = jnp.sum(degp_ref[...], axis=0) + 1.0
    dinv = lax.rsqrt(deg)
    y0 = jnp.dot(x_ref[...], w0_ref[...], preferred_element_type=jnp.float32)
    u0_ref[...] = y0 * dinv[:, None]
    d2b_ref[...] = jnp.broadcast_to((dinv * dinv)[:, None], (BLK, HC))
    db_ref[...] = jnp.broadcast_to(dinv[:, None], (BLK, HC))


def _sigmoid(v):
    return 1.0 / (1.0 + jnp.exp(-v))


def _softplus(v):
    return jnp.maximum(v, 0.0) + jnp.log(1.0 + jnp.exp(-jnp.abs(v)))


def _k5_body(g_ref, wcat_ref, fblk_ref, zm_ref, zs_ref, zp_ref, za_ref):
    g = g_ref[...]
    h = _sigmoid(jnp.dot(g, wcat_ref[...], preferred_element_type=jnp.float32))
    z = jnp.dot(h, fblk_ref[...], preferred_element_type=jnp.float32)
    zm_ref[...] = z[:, 0:32]
    zs_ref[...] = z[:, 32:64]
    zp_ref[...] = z[:, 64:96]
    za_ref[...] = _softplus(z[:, 96:128])


def kernel(x, edge_index, W0, Wm, Ws, Wp, Wa, Fm, Fs, Fp, Fa):
    src = edge_index[0]
    dst = edge_index[1]
    src_t = jnp.pad(src.reshape(NS, EPT), ((0, 0), (0, EPT_PAD - EPT)),
                    constant_values=PADROW).reshape(NS, NCHUNK, CHUNK)
    dst_t = jnp.pad(dst.reshape(NS, EPT), ((0, 0), (0, EPT_PAD - EPT)),
                    constant_values=PADROW).reshape(NS, NCHUNK, CHUNK)
    dst_f = dst_t.reshape(NW, EPT_PAD // 2)
    x_pad = jnp.pad(x, ((0, NPAD - N), (0, 0)))
    wcat = jnp.concatenate([Wm, Ws, Wp, Wa], axis=1)
    fblk = jax.scipy.linalg.block_diag(Fm, Fs, Fp, Fa)

    degp = _deg_kernel(dst_f)

    u0, d2b, db = pl.pallas_call(
        _k2_body,
        grid=(GRID,),
        in_specs=[
            pl.BlockSpec((BLK, D_IN), lambda i: (i, 0)),
            pl.BlockSpec((D_IN, H1), lambda i: (0, 0)),
            pl.BlockSpec((NW, BLK), lambda i: (0, i)),
        ],
        out_specs=[
            pl.BlockSpec((BLK, H1), lambda i: (i, 0)),
            pl.BlockSpec((BLK, HC), lambda i: (i, 0)),
            pl.BlockSpec((BLK, HC), lambda i: (i, 0)),
        ],
        out_shape=[
            jax.ShapeDtypeStruct((NPAD, H1), jnp.float32),
            jax.ShapeDtypeStruct((NPAD, HC), jnp.float32),
            jax.ShapeDtypeStruct((NPAD, HC), jnp.float32),
        ],
    )(x_pad, W0, degp)

    g1 = _gcn_kernel(u0, d2b, db, src_t, dst_t)

    zm, zs, zp, za = pl.pallas_call(
        _k5_body,
        grid=(GRID,),
        in_specs=[
            pl.BlockSpec((BLK, H1), lambda i: (i, 0)),
            pl.BlockSpec((H1, 128), lambda i: (0, 0)),
            pl.BlockSpec((128, 128), lambda i: (0, 0)),
        ],
        out_specs=[pl.BlockSpec((BLK, 32), lambda i: (i, 0))] * 4,
        out_shape=[jax.ShapeDtypeStruct((NPAD, 32), jnp.float32)] * 4,
    )(g1, wcat, fblk)

    return jnp.stack([zm, zs, zp, za])[:, :N]


# speedup vs baseline: 1.1297x; 1.0544x over previous
import functools

import jax
import jax.numpy as jnp
from jax import lax
from jax.experimental import pallas as pl
from jax.experimental.pallas import tpu as pltpu
from jax.experimental.pallas import tpu_sc as plsc

N = 10000
E = 320000
D_IN = 128
H1 = 64
HC = 32

NC = 2
NS = 16
NPAD = 10240
ROWS_PER_TILE = NPAD // NS
RBLK = 128
NRBLK = ROWS_PER_TILE // RBLK

EPT = E // NS
CHUNK = 128
NBUF = 2
NCHUNK = 160
EPT_PAD = NCHUNK * CHUNK
PADROW = NPAD - 1

_mesh = plsc.VectorSubcoreMesh(core_axis_name="c", subcore_axis_name="s")
_sc_params = pltpu.CompilerParams(
    needs_layout_passes=False, use_tc_tiling_on_sc=False)


def _rsqrt16(x):
    i = plsc.bitcast(x, jnp.int32)
    i = jnp.int32(0x5F3759DF) - lax.shift_right_arithmetic(i, 1)
    y = plsc.bitcast(i, jnp.float32)
    hx = 0.5 * x
    for _ in range(3):
        y = y * (1.5 - hx * y * y)
    return y


@functools.partial(
    pl.kernel,
    out_type=jax.ShapeDtypeStruct((NPAD, H1), jnp.float32),
    mesh=_mesh,
    compiler_params=_sc_params,
    scratch_types=[
        pltpu.VMEM((NCHUNK, CHUNK), jnp.int32),
        pltpu.VMEM((NCHUNK, CHUNK), jnp.int32),
        pltpu.VMEM((NPAD,), jnp.float32),
        pltpu.VMEM((NS, ROWS_PER_TILE), jnp.float32),
        pltpu.VMEM((ROWS_PER_TILE,), jnp.float32),
        pltpu.VMEM((ROWS_PER_TILE,), jnp.float32),
    ] + [pltpu.VMEM((CHUNK, HC), jnp.float32) for _ in range(NBUF)]
      + [pltpu.SemaphoreType.DMA for _ in range(2 * NBUF)]
      + [pltpu.VMEM_SHARED((NPAD, HC), jnp.float32),
         pltpu.VMEM_SHARED((NPAD, HC), jnp.float32),
         pltpu.VMEM_SHARED((NS, NPAD), jnp.float32)],
)
def _gcn_kernel(y0_hbm, src_hbm, dst_hbm, g1_hbm, src_v, dst_v, hist_v,
                hsum_v, dinv_v, dinv2_v, *scr):
    rows = scr[0:NBUF]
    gsem = scr[NBUF:2 * NBUF]
    ssem = scr[2 * NBUF:3 * NBUF]
    acc = scr[3 * NBUF]
    u_s = scr[3 * NBUF + 1]
    hist_s = scr[3 * NBUF + 2]
    c = lax.axis_index("c")
    s = lax.axis_index("s")
    base = s * ROWS_PER_TILE
    cols = pl.ds(c * HC, HC)

    pltpu.sync_copy(src_hbm.at[s], src_v)
    pltpu.sync_copy(dst_hbm.at[s], dst_v)

    def zero_body(i, _):
        hist_v[pl.ds(i * 16, 16)] = jnp.zeros((16,), jnp.float32)
        return 0

    lax.fori_loop(0, NPAD // 16, zero_body, 0)
    ones = jnp.ones((16,), jnp.float32)

    def hist_body(g, _):
        for o in range(CHUNK // 16):
            idx = dst_v[g, pl.ds(o * 16, 16)]
            plsc.addupdate_scatter(hist_v, [idx], ones)
        return 0

    lax.fori_loop(0, NCHUNK, hist_body, 0)
    pltpu.sync_copy(hist_v, hist_s.at[s])
    plsc.subcore_barrier()

    for t in range(NS):
        pltpu.sync_copy(hist_s.at[t, pl.ds(base, ROWS_PER_TILE)],
                        hsum_v.at[t])

    def dinv_body(g, _):
        sl = pl.ds(g * 16, 16)
        deg = jnp.ones((16,), jnp.float32)
        for t in range(NS):
            deg = deg + hsum_v[t, sl]
        d = _rsqrt16(deg)
        dinv_v[sl] = d
        dinv2_v[sl] = d * d
        return 0

    lax.fori_loop(0, ROWS_PER_TILE // 16, dinv_body, 0)

    def scale_rows(dst_ref, src_ref, d_ref, kblk):
        def body(g, _):
            dv = d_ref[pl.ds(kblk * RBLK + g * 16, 16)]
            for l in range(16):
                r = g * 16 + l
                for o in range(HC // 16):
                    sl = pl.ds(o * 16, 16)
                    dst_ref[r, sl] = src_ref[r, sl] * dv[l]
            return 0

        lax.fori_loop(0, RBLK // 16, body, 0)

    for k in range(NRBLK):
        rk = pl.ds(base + k * RBLK, RBLK)
        pltpu.sync_copy(y0_hbm.at[rk, cols], rows[0])
        scale_rows(rows[1], rows[0], dinv_v, k)
        pltpu.sync_copy(rows[1], u_s.at[rk])
        pltpu.sync_copy(rows[1], acc.at[rk])
    plsc.subcore_barrier()

    def edge_pass():
        for b in range(NBUF):
            pltpu.async_copy(u_s.at[src_v.at[b]], rows[b], gsem[b])

        def outer(t, _):
            cbase = t * NBUF
            for b in range(NBUF):
                j = cbase + b
                pltpu.make_async_copy(
                    u_s.at[src_v.at[j]], rows[b], gsem[b]).wait()
                pltpu.async_copy(rows[b], acc.at[dst_v.at[j]], ssem[b],
                                 add=True)
            for b in range(NBUF):
                j = cbase + b

                @pl.when(j + NBUF < NCHUNK)
                def _():
                    pltpu.make_async_copy(
                        rows[b], acc.at[dst_v.at[j]], ssem[b]).wait()
                    pltpu.async_copy(
                        u_s.at[src_v.at[j + NBUF]], rows[b], gsem[b])
            return 0

        lax.fori_loop(0, NCHUNK // NBUF, outer, 0)
        for b in range(NBUF):
            pltpu.make_async_copy(
                rows[b], acc.at[dst_v.at[NCHUNK - NBUF + b]], ssem[b]).wait()

    edge_pass()
    plsc.subcore_barrier()

    for k in range(NRBLK):
        rk = pl.ds(base + k * RBLK, RBLK)
        pltpu.sync_copy(acc.at[rk], rows[0])
        scale_rows(rows[1], rows[0], dinv2_v, k)
        pltpu.sync_copy(rows[1], u_s.at[rk])
        pltpu.sync_copy(rows[1], acc.at[rk])
    plsc.subcore_barrier()

    edge_pass()
    plsc.subcore_barrier()

    for k in range(NRBLK):
        rk = pl.ds(base + k * RBLK, RBLK)
        pltpu.sync_copy(acc.at[rk], rows[0])
        scale_rows(rows[1], rows[0], dinv_v, k)
        pltpu.sync_copy(rows[1], g1_hbm.at[rk, cols])


BLK = 512
GRID = NPAD // BLK


def _k2_body(x_ref, w0_ref, y0_ref):
    y0_ref[...] = jnp.dot(x_ref[...], w0_ref[...],
                          preferred_element_type=jnp.float32)


def _sigmoid(v):
    return 1.0 / (1.0 + jnp.exp(-v))


def _softplus(v):
    return jnp.maximum(v, 0.0) + jnp.log(1.0 + jnp.exp(-jnp.abs(v)))


def _k5_body(g_ref, wcat_ref, fblk_ref, zm_ref, zs_ref, zp_ref, za_ref):
    g = g_ref[...]
    h = _sigmoid(jnp.dot(g, wcat_ref[...], preferred_element_type=jnp.float32))
    z = jnp.dot(h, fblk_ref[...], preferred_element_type=jnp.float32)
    zm_ref[...] = z[:, 0:32]
    zs_ref[...] = z[:, 32:64]
    zp_ref[...] = z[:, 64:96]
    za_ref[...] = _softplus(z[:, 96:128])


def kernel(x, edge_index, W0, Wm, Ws, Wp, Wa, Fm, Fs, Fp, Fa):
    src = edge_index[0]
    dst = edge_index[1]
    src_t = jnp.pad(src.reshape(NS, EPT), ((0, 0), (0, EPT_PAD - EPT)),
                    constant_values=PADROW).reshape(NS, NCHUNK, CHUNK)
    dst_t = jnp.pad(dst.reshape(NS, EPT), ((0, 0), (0, EPT_PAD - EPT)),
                    constant_values=PADROW).reshape(NS, NCHUNK, CHUNK)
    x_pad = jnp.pad(x, ((0, NPAD - N), (0, 0)))
    wcat = jnp.concatenate([Wm, Ws, Wp, Wa], axis=1)
    fblk = jax.scipy.linalg.block_diag(Fm, Fs, Fp, Fa)

    y0 = pl.pallas_call(
        _k2_body,
        grid=(GRID,),
        in_specs=[
            pl.BlockSpec((BLK, D_IN), lambda i: (i, 0)),
            pl.BlockSpec((D_IN, H1), lambda i: (0, 0)),
        ],
        out_specs=pl.BlockSpec((BLK, H1), lambda i: (i, 0)),
        out_shape=jax.ShapeDtypeStruct((NPAD, H1), jnp.float32),
    )(x_pad, W0)

    g1 = _gcn_kernel(y0, src_t, dst_t)

    zm, zs, zp, za = pl.pallas_call(
        _k5_body,
        grid=(GRID,),
        in_specs=[
            pl.BlockSpec((BLK, H1), lambda i: (i, 0)),
            pl.BlockSpec((H1, 128), lambda i: (0, 0)),
            pl.BlockSpec((128, 128), lambda i: (0, 0)),
        ],
        out_specs=[pl.BlockSpec((BLK, 32), lambda i: (i, 0))] * 4,
        out_shape=[jax.ShapeDtypeStruct((NPAD, 32), jnp.float32)] * 4,
    )(g1, wcat, fblk)

    return jnp.stack([zm, zs, zp, za])[:, :N]


# speedup vs baseline: 1.2336x; 1.0921x over previous
import functools

import jax
import jax.numpy as jnp
from jax import lax
from jax.experimental import pallas as pl
from jax.experimental.pallas import tpu as pltpu
from jax.experimental.pallas import tpu_sc as plsc

N = 10000
E = 320000
D_IN = 128
H1 = 64
HC = 32

NC = 2
NS = 16
NPAD = 10240
ROWS_PER_TILE = NPAD // NS
RBLK = 80
NRBLK = ROWS_PER_TILE // RBLK

EPT = E // NS
CHUNK = 80
NBUF = 2
NCHUNK = EPT // CHUNK

_mesh = plsc.VectorSubcoreMesh(core_axis_name="c", subcore_axis_name="s")
_sc_params = pltpu.CompilerParams(
    needs_layout_passes=False, use_tc_tiling_on_sc=False)


def _rsqrt16(x):
    i = plsc.bitcast(x, jnp.int32)
    i = jnp.int32(0x5F3759DF) - lax.shift_right_arithmetic(i, 1)
    y = plsc.bitcast(i, jnp.float32)
    hx = 0.5 * x
    for _ in range(3):
        y = y * (1.5 - hx * y * y)
    return y


@functools.partial(
    pl.kernel,
    out_type=jax.ShapeDtypeStruct((NPAD, H1), jnp.float32),
    mesh=_mesh,
    compiler_params=_sc_params,
    scratch_types=[
        pltpu.VMEM((NCHUNK, CHUNK), jnp.int32),
        pltpu.VMEM((NCHUNK, CHUNK), jnp.int32),
        pltpu.VMEM((NPAD,), jnp.float32),
        pltpu.VMEM((NS, ROWS_PER_TILE), jnp.float32),
        pltpu.VMEM((ROWS_PER_TILE,), jnp.float32),
        pltpu.VMEM((ROWS_PER_TILE,), jnp.float32),
    ] + [pltpu.VMEM((CHUNK, HC), jnp.float32) for _ in range(NBUF)]
      + [pltpu.SemaphoreType.DMA for _ in range(2 * NBUF)]
      + [pltpu.VMEM_SHARED((NPAD, HC), jnp.float32),
         pltpu.VMEM_SHARED((NPAD, HC), jnp.float32),
         pltpu.VMEM_SHARED((NS, NPAD), jnp.float32)],
)
def _gcn_kernel(y0_hbm, src_hbm, dst_hbm, g1_hbm, src_v, dst_v, hist_v,
                hsum_v, dinv_v, dinv2_v, *scr):
    rows = scr[0:NBUF]
    gsem = scr[NBUF:2 * NBUF]
    ssem = scr[2 * NBUF:3 * NBUF]
    acc = scr[3 * NBUF]
    u_s = scr[3 * NBUF + 1]
    hist_s = scr[3 * NBUF + 2]
    c = lax.axis_index("c")
    s = lax.axis_index("s")
    base = s * ROWS_PER_TILE
    cols = pl.ds(c * HC, HC)

    pltpu.sync_copy(src_hbm.at[s], src_v)
    pltpu.sync_copy(dst_hbm.at[s], dst_v)

    def zero_body(i, _):
        hist_v[pl.ds(i * 16, 16)] = jnp.zeros((16,), jnp.float32)
        return 0

    lax.fori_loop(0, NPAD // 16, zero_body, 0)
    ones = jnp.ones((16,), jnp.float32)

    def hist_body(g, _):
        for o in range(CHUNK // 16):
            idx = dst_v[g, pl.ds(o * 16, 16)]
            plsc.addupdate_scatter(hist_v, [idx], ones)
        return 0

    lax.fori_loop(0, NCHUNK, hist_body, 0)
    pltpu.sync_copy(hist_v, hist_s.at[s])
    plsc.subcore_barrier()

    for t in range(NS):
        pltpu.sync_copy(hist_s.at[t, pl.ds(base, ROWS_PER_TILE)],
                        hsum_v.at[t])

    def dinv_body(g, _):
        sl = pl.ds(g * 16, 16)
        deg = jnp.ones((16,), jnp.float32)
        for t in range(NS):
            deg = deg + hsum_v[t, sl]
        d = _rsqrt16(deg)
        dinv_v[sl] = d
        dinv2_v[sl] = d * d
        return 0

    lax.fori_loop(0, ROWS_PER_TILE // 16, dinv_body, 0)

    def scale_rows(dst_ref, src_ref, d_ref, kblk):
        def body(g, _):
            dv = d_ref[pl.ds(kblk * RBLK + g * 16, 16)]
            for l in range(16):
                r = g * 16 + l
                for o in range(HC // 16):
                    sl = pl.ds(o * 16, 16)
                    dst_ref[r, sl] = src_ref[r, sl] * dv[l]
            return 0

        lax.fori_loop(0, RBLK // 16, body, 0)

    for k in range(NRBLK):
        rk = pl.ds(base + k * RBLK, RBLK)
        pltpu.sync_copy(y0_hbm.at[rk, cols], rows[0])
        scale_rows(rows[1], rows[0], dinv_v, k)
        pltpu.sync_copy(rows[1], u_s.at[rk])
        pltpu.sync_copy(rows[1], acc.at[rk])
    plsc.subcore_barrier()

    def edge_pass():
        for b in range(NBUF):
            pltpu.async_copy(u_s.at[src_v.at[b]], rows[b], gsem[b])

        def outer(t, _):
            cbase = t * NBUF
            for b in range(NBUF):
                j = cbase + b
                pltpu.make_async_copy(
                    u_s.at[src_v.at[j]], rows[b], gsem[b]).wait()
                pltpu.async_copy(rows[b], acc.at[dst_v.at[j]], ssem[b],
                                 add=True)
            for b in range(NBUF):
                j = cbase + b

                @pl.when(j + NBUF < NCHUNK)
                def _():
                    pltpu.make_async_copy(
                        rows[b], acc.at[dst_v.at[j]], ssem[b]).wait()
                    pltpu.async_copy(
                        u_s.at[src_v.at[j + NBUF]], rows[b], gsem[b])
            return 0

        lax.fori_loop(0, NCHUNK // NBUF, outer, 0)
        for b in range(NBUF):
            pltpu.make_async_copy(
                rows[b], acc.at[dst_v.at[NCHUNK - NBUF + b]], ssem[b]).wait()

    edge_pass()
    plsc.subcore_barrier()

    for k in range(NRBLK):
        rk = pl.ds(base + k * RBLK, RBLK)
        pltpu.sync_copy(acc.at[rk], rows[0])
        scale_rows(rows[1], rows[0], dinv2_v, k)
        pltpu.sync_copy(rows[1], u_s.at[rk])
        pltpu.sync_copy(rows[1], acc.at[rk])
    plsc.subcore_barrier()

    edge_pass()
    plsc.subcore_barrier()

    for k in range(NRBLK):
        rk = pl.ds(base + k * RBLK, RBLK)
        pltpu.sync_copy(acc.at[rk], rows[0])
        scale_rows(rows[1], rows[0], dinv_v, k)
        pltpu.sync_copy(rows[1], g1_hbm.at[rk, cols])


BLK = 512
GRID = NPAD // BLK


def _k2_body(x_ref, w0_ref, y0_ref):
    y0_ref[...] = jnp.dot(x_ref[...], w0_ref[...],
                          preferred_element_type=jnp.float32)


def _k5_body2(g_ref, wcat_ref, fblk_ref, out_ref):
    g = g_ref[...]
    h = _sigmoid(jnp.dot(g, wcat_ref[...], preferred_element_type=jnp.float32))
    z = jnp.dot(h, fblk_ref[...], preferred_element_type=jnp.float32)
    out_ref[...] = jnp.stack(
        [z[:, 0:32], z[:, 32:64], z[:, 64:96], _softplus(z[:, 96:128])],
        axis=0)


def _sigmoid(v):
    return 1.0 / (1.0 + jnp.exp(-v))


def _softplus(v):
    return jnp.maximum(v, 0.0) + jnp.log(1.0 + jnp.exp(-jnp.abs(v)))


def _k5_body(g_ref, wcat_ref, fblk_ref, zm_ref, zs_ref, zp_ref, za_ref):
    g = g_ref[...]
    h = _sigmoid(jnp.dot(g, wcat_ref[...], preferred_element_type=jnp.float32))
    z = jnp.dot(h, fblk_ref[...], preferred_element_type=jnp.float32)
    zm_ref[...] = z[:, 0:32]
    zs_ref[...] = z[:, 32:64]
    zp_ref[...] = z[:, 64:96]
    za_ref[...] = _softplus(z[:, 96:128])


def kernel(x, edge_index, W0, Wm, Ws, Wp, Wa, Fm, Fs, Fp, Fa):
    src = edge_index[0]
    dst = edge_index[1]
    src_t = src.reshape(NS, NCHUNK, CHUNK)
    dst_t = dst.reshape(NS, NCHUNK, CHUNK)
    x_pad = jnp.pad(x, ((0, NPAD - N), (0, 0)))
    wcat = jnp.concatenate([Wm, Ws, Wp, Wa], axis=1)
    fblk = jax.scipy.linalg.block_diag(Fm, Fs, Fp, Fa)

    y0 = pl.pallas_call(
        _k2_body,
        out_shape=jax.ShapeDtypeStruct((NPAD, H1), jnp.float32),
    )(x_pad, W0)

    g1 = _gcn_kernel(y0, src_t, dst_t)

    out = pl.pallas_call(
        _k5_body2,
        grid=(GRID,),
        in_specs=[
            pl.BlockSpec((BLK, H1), lambda i: (i, 0)),
            pl.BlockSpec((H1, 128), lambda i: (0, 0)),
            pl.BlockSpec((128, 128), lambda i: (0, 0)),
        ],
        out_specs=pl.BlockSpec((4, BLK, 32), lambda i: (0, i, 0)),
        out_shape=jax.ShapeDtypeStruct((4, N, 32), jnp.float32),
    )(g1, wcat, fblk)

    return out
